# baseline (speedup 1.0000x reference)
import jax
import jax.numpy as jnp
from jax.experimental import pallas as pl
from jax.experimental.pallas import tpu as pltpu

_B, _H, _S, _D = 32, 8, 2048, 64


def _update_body(inv_ref, pos_ref, k_hbm, v_hbm, nk_hbm, nv_hbm, out_hbm,
                 copy_sems, scat_sems):
    copies = []
    for b in range(_B):
        ck = pltpu.make_async_copy(k_hbm.at[b], out_hbm.at[0, b],
                                   copy_sems.at[2 * b])
        cv = pltpu.make_async_copy(v_hbm.at[b], out_hbm.at[1, b],
                                   copy_sems.at[2 * b + 1])
        ck.start()
        cv.start()
        copies.append(ck)
        copies.append(cv)
    for c in copies:
        c.wait()
    scats = []
    for b in range(_B):
        src = inv_ref[b]
        p = pos_ref[b]
        sk = pltpu.make_async_copy(nk_hbm.at[src, :, :, :],
                                   out_hbm.at[0, b, :, pl.ds(p, 1), :],
                                   scat_sems.at[2 * b])
        sv = pltpu.make_async_copy(nv_hbm.at[src, :, :, :],
                                   out_hbm.at[1, b, :, pl.ds(p, 1), :],
                                   scat_sems.at[2 * b + 1])
        sk.start()
        sv.start()
        scats.append(sk)
        scats.append(sv)
    for s in scats:
        s.wait()


def kernel(k_cache, v_cache, new_k, new_v, seq_ids, position_ids):
    b, h, s, d = k_cache.shape
    inv = jnp.argsort(seq_ids).astype(jnp.int32)
    pos = position_ids[inv, 0].astype(jnp.int32)
    return pl.pallas_call(
        _update_body,
        grid=(),
        in_specs=[
            pl.BlockSpec(memory_space=pltpu.SMEM),
            pl.BlockSpec(memory_space=pltpu.SMEM),
            pl.BlockSpec(memory_space=pl.ANY),
            pl.BlockSpec(memory_space=pl.ANY),
            pl.BlockSpec(memory_space=pl.ANY),
            pl.BlockSpec(memory_space=pl.ANY),
        ],
        out_specs=pl.BlockSpec(memory_space=pl.ANY),
        out_shape=jax.ShapeDtypeStruct((2, b, h, s, d), k_cache.dtype),
        scratch_shapes=[
            pltpu.SemaphoreType.DMA((2 * _B,)),
            pltpu.SemaphoreType.DMA((2 * _B,)),
        ],
    )(inv, pos, k_cache, v_cache, new_k, new_v)


# speedup vs baseline: 12.5275x; 12.5275x over previous
import jax
import jax.numpy as jnp
from jax import lax
from jax.experimental import pallas as pl
from jax.experimental.pallas import tpu as pltpu

_B, _H, _S, _D = 32, 8, 2048, 64
_R = _S * _D // 128
_SB = 256
_NSB = _R // _SB


def _update_body(inv_ref, pos_ref, k_ref, v_ref, nk_ref, nv_ref, out_ref):
    b = pl.program_id(0)
    sb = pl.program_id(1)
    out_ref[0] = k_ref[...]
    out_ref[1] = v_ref[...]
    p = pos_ref[b]
    prow = p // 2
    pcol = (p % 2) * 64
    local = prow - sb * _SB

    @pl.when((prow >= sb * _SB) & (prow < (sb + 1) * _SB))
    def _():
        rows = lax.broadcasted_iota(jnp.int32, (1, _H, _SB, 128), 2)
        cols = lax.broadcasted_iota(jnp.int32, (1, _H, _SB, 128), 3)
        sel = (rows == local) & (cols >= pcol) & (cols < pcol + 64)
        nk = nk_ref[0]
        nv = nv_ref[0]
        repk = jnp.concatenate([nk, nk], axis=-1)[None, :, None, :]
        repv = jnp.concatenate([nv, nv], axis=-1)[None, :, None, :]
        out_ref[0] = jnp.where(sel, repk, k_ref[...])
        out_ref[1] = jnp.where(sel, repv, v_ref[...])


def kernel(k_cache, v_cache, new_k, new_v, seq_ids, position_ids):
    b, h, s, d = k_cache.shape
    inv = jnp.argsort(seq_ids).astype(jnp.int32)
    pos = position_ids[inv, 0].astype(jnp.int32)
    k3 = k_cache.reshape(b, h, _R, 128)
    v3 = v_cache.reshape(b, h, _R, 128)
    nk = new_k.reshape(b, h, d)
    nv = new_v.reshape(b, h, d)
    grid_spec = pltpu.PrefetchScalarGridSpec(
        num_scalar_prefetch=2,
        grid=(b, _NSB),
        in_specs=[
            pl.BlockSpec((1, h, _SB, 128), lambda i, j, inv_r, pos_r: (i, 0, j, 0)),
            pl.BlockSpec((1, h, _SB, 128), lambda i, j, inv_r, pos_r: (i, 0, j, 0)),
            pl.BlockSpec((1, h, d), lambda i, j, inv_r, pos_r: (inv_r[i], 0, 0)),
            pl.BlockSpec((1, h, d), lambda i, j, inv_r, pos_r: (inv_r[i], 0, 0)),
        ],
        out_specs=pl.BlockSpec(
            (2, 1, h, _SB, 128), lambda i, j, inv_r, pos_r: (0, i, 0, j, 0)),
    )
    out = pl.pallas_call(
        _update_body,
        grid_spec=grid_spec,
        out_shape=jax.ShapeDtypeStruct((2, b, h, _R, 128), k_cache.dtype),
    )(inv, pos, k3, v3, nk, nv)
    return out.reshape(2, b, h, s, d)


# speedup vs baseline: 12.7204x; 1.0154x over previous
import jax
import jax.numpy as jnp
from jax import lax
from jax.experimental import pallas as pl
from jax.experimental.pallas import tpu as pltpu

_B, _H, _S, _D = 32, 8, 2048, 64
_R = _S * _D // 128
_SB = 512
_NSB = _R // _SB


def _update_body(inv_ref, pos_ref, k_ref, v_ref, nk_ref, nv_ref, out_ref):
    b = pl.program_id(0)
    sb = pl.program_id(1)
    out_ref[0] = k_ref[...]
    out_ref[1] = v_ref[...]
    p = pos_ref[b]
    prow = p // 2
    pcol = (p % 2) * 64
    local = prow - sb * _SB

    @pl.when((prow >= sb * _SB) & (prow < (sb + 1) * _SB))
    def _():
        rows = lax.broadcasted_iota(jnp.int32, (1, _H, _SB, 128), 2)
        cols = lax.broadcasted_iota(jnp.int32, (1, _H, _SB, 128), 3)
        sel = (rows == local) & (cols >= pcol) & (cols < pcol + 64)
        nk = nk_ref[0]
        nv = nv_ref[0]
        repk = jnp.concatenate([nk, nk], axis=-1)[None, :, None, :]
        repv = jnp.concatenate([nv, nv], axis=-1)[None, :, None, :]
        out_ref[0] = jnp.where(sel, repk, k_ref[...])
        out_ref[1] = jnp.where(sel, repv, v_ref[...])


def kernel(k_cache, v_cache, new_k, new_v, seq_ids, position_ids):
    b, h, s, d = k_cache.shape
    inv = jnp.argsort(seq_ids).astype(jnp.int32)
    pos = position_ids[inv, 0].astype(jnp.int32)
    k3 = k_cache.reshape(b, h, _R, 128)
    v3 = v_cache.reshape(b, h, _R, 128)
    nk = new_k.reshape(b, h, d)
    nv = new_v.reshape(b, h, d)
    grid_spec = pltpu.PrefetchScalarGridSpec(
        num_scalar_prefetch=2,
        grid=(b, _NSB),
        in_specs=[
            pl.BlockSpec((1, h, _SB, 128), lambda i, j, inv_r, pos_r: (i, 0, j, 0)),
            pl.BlockSpec((1, h, _SB, 128), lambda i, j, inv_r, pos_r: (i, 0, j, 0)),
            pl.BlockSpec((1, h, d), lambda i, j, inv_r, pos_r: (inv_r[i], 0, 0)),
            pl.BlockSpec((1, h, d), lambda i, j, inv_r, pos_r: (inv_r[i], 0, 0)),
        ],
        out_specs=pl.BlockSpec(
            (2, 1, h, _SB, 128), lambda i, j, inv_r, pos_r: (0, i, 0, j, 0)),
    )
    out = pl.pallas_call(
        _update_body,
        grid_spec=grid_spec,
        out_shape=jax.ShapeDtypeStruct((2, b, h, _R, 128), k_cache.dtype),
    )(inv, pos, k3, v3, nk, nv)
    return out.reshape(2, b, h, s, d)
